# baseline (speedup 1.0000x reference)
import functools

import jax
import jax.numpy as jnp
from jax import lax
from jax.experimental import pallas as pl
from jax.experimental.pallas import tpu as pltpu
from jax.experimental.pallas import tpu_sc as plsc

N = 5000
D = 128
E = 32768
BATCH = 16

NC = 2
NS = 16
BPC = BATCH // NC

ROWS_PER_TILE = 320
NPAD = ROWS_PER_TILE * NS
EPT = E // NS
CHUNK = 128
NCHUNKS = EPT // CHUNK

_sc_mesh = plsc.VectorSubcoreMesh(core_axis_name="c", subcore_axis_name="s")


@functools.partial(
    pl.kernel,
    out_type=[
        jax.ShapeDtypeStruct((BATCH, NPAD, D), jnp.float32),
        jax.ShapeDtypeStruct((NPAD, D), jnp.float32),
    ],
    mesh=_sc_mesh,
    scratch_types=[
        pltpu.VMEM((NCHUNKS, CHUNK), jnp.int32),
        pltpu.VMEM((NCHUNKS, CHUNK), jnp.int32),
        pltpu.VMEM((4, CHUNK, D), jnp.float32),
        pltpu.VMEM((ROWS_PER_TILE // 5, D), jnp.float32),
        pltpu.VMEM_SHARED((NPAD, D), jnp.float32),
        pltpu.SemaphoreType.DMA,
        pltpu.SemaphoreType.DMA,
        pltpu.SemaphoreType.DMA,
        pltpu.SemaphoreType.DMA,
    ],
)
def _sc_scatter(x_hbm, src_hbm, dst_hbm, agg_hbm, deg_hbm,
                srcv, dstv, rows, zbuf, agg_sh, sem0, sem1, sem2, sem3):
    c = lax.axis_index("c")
    s = lax.axis_index("s")
    my = pl.ds(s * ROWS_PER_TILE, ROWS_PER_TILE)

    zero16 = jnp.zeros((16,), jnp.float32)

    def _zrow(i, _):
        for l in range(D // 16):
            zbuf[i, pl.ds(l * 16, 16)] = zero16
        return 0

    lax.fori_loop(0, ROWS_PER_TILE // 5, _zrow, 0)

    def _zero_my_slice():
        for z in range(5):
            pltpu.sync_copy(
                zbuf,
                agg_sh.at[pl.ds(s * ROWS_PER_TILE + z * (ROWS_PER_TILE // 5),
                                ROWS_PER_TILE // 5)])

    pltpu.sync_copy(src_hbm.at[pl.ds(s * NCHUNKS, NCHUNKS)], srcv)
    pltpu.sync_copy(dst_hbm.at[pl.ds(s * NCHUNKS, NCHUNKS)], dstv)

    @pl.when(c == 0)
    def _deg():
        one16 = jnp.full((16,), 1.0, jnp.float32)

        def _orow(i, _):
            for l in range(D // 16):
                rows[0, i, pl.ds(l * 16, 16)] = one16
            return 0

        lax.fori_loop(0, CHUNK, _orow, 0)
        _zero_my_slice()
        plsc.subcore_barrier()
        for j in range(NCHUNKS):
            pltpu.sync_copy(rows.at[0], agg_sh.at[dstv.at[j]], add=True)
        plsc.subcore_barrier()
        pltpu.sync_copy(agg_sh.at[my], deg_hbm.at[my])

    base0 = c * (BPC * N)

    def _shift(i, _):
        for l in range(CHUNK // 16):
            sl = pl.ds(l * 16, 16)
            srcv[i, sl] = srcv[i, sl] + base0
        return 0

    lax.fori_loop(0, NCHUNKS, _shift, 0)

    NBUF = 4
    sems = (sem0, sem1, sem2, sem3)

    def _batch(b, _):
        _zero_my_slice()
        plsc.subcore_barrier()
        gat = [None] * NCHUNKS
        scat = [None] * NCHUNKS
        for j in range(NBUF - 1):
            gat[j] = pltpu.async_copy(
                x_hbm.at[srcv.at[j]], rows.at[j % NBUF], sems[j % NBUF])
        for j in range(NCHUNKS):
            if j - 1 >= 0:
                scat[j - 1].wait()
            if j + NBUF - 1 < NCHUNKS:
                jn = j + NBUF - 1
                gat[jn] = pltpu.async_copy(
                    x_hbm.at[srcv.at[jn]], rows.at[jn % NBUF], sems[jn % NBUF])
            gat[j].wait()
            scat[j] = pltpu.async_copy(
                rows.at[j % NBUF], agg_sh.at[dstv.at[j]], sems[j % NBUF],
                add=True)
        scat[NCHUNKS - 1].wait()
        plsc.subcore_barrier()
        bg = c * BPC + b
        pltpu.sync_copy(agg_sh.at[my], agg_hbm.at[bg, my])

        def _bump(i, _):
            for l in range(CHUNK // 16):
                sl = pl.ds(l * 16, 16)
                srcv[i, sl] = srcv[i, sl] + N
            return 0

        lax.fori_loop(0, NCHUNKS, _bump, 0)
        return 0

    lax.fori_loop(0, BPC, _batch, 0)


BN = 1000


def _dense_body(x_ref, agg_ref, deg_ref, ws_ref, wn_ref, b_ref, g_ref, be_ref,
                o_ref):
    xb = x_ref[0]
    inv = 1.0 / jnp.maximum(deg_ref[...], 1.0)
    neigh = agg_ref[0] * inv
    out = (jnp.dot(xb, ws_ref[...], preferred_element_type=jnp.float32)
           + jnp.dot(neigh, wn_ref[...], preferred_element_type=jnp.float32)
           + b_ref[...])
    mu = jnp.mean(out, axis=-1, keepdims=True)
    var = jnp.mean((out - mu) ** 2, axis=-1, keepdims=True)
    out = (out - mu) * lax.rsqrt(var + 1e-5) * g_ref[...] + be_ref[...]
    o_ref[0] = jnp.maximum(out, 0.0)


_dense = pl.pallas_call(
    _dense_body,
    grid=(BATCH, N // BN),
    in_specs=[
        pl.BlockSpec((1, BN, D), lambda b, j: (b, j, 0)),
        pl.BlockSpec((1, BN, D), lambda b, j: (b, j, 0)),
        pl.BlockSpec((BN, D), lambda b, j: (j, 0)),
        pl.BlockSpec((D, D), lambda b, j: (0, 0)),
        pl.BlockSpec((D, D), lambda b, j: (0, 0)),
        pl.BlockSpec((1, D), lambda b, j: (0, 0)),
        pl.BlockSpec((1, D), lambda b, j: (0, 0)),
        pl.BlockSpec((1, D), lambda b, j: (0, 0)),
    ],
    out_specs=pl.BlockSpec((1, BN, D), lambda b, j: (b, j, 0)),
    out_shape=jax.ShapeDtypeStruct((BATCH, N, D), jnp.float32),
    compiler_params=pltpu.CompilerParams(
        dimension_semantics=("parallel", "parallel")),
)


def kernel(x, edge_index, batch_size, W_self, W_neigh, bias, gamma, beta):
    x_flat = x.reshape(BATCH * N, D)
    src2d = edge_index[0].reshape(E // CHUNK, CHUNK)
    dst2d = edge_index[1].reshape(E // CHUNK, CHUNK)
    agg_pad, deg_pad = _sc_scatter(x_flat, src2d, dst2d)
    out = _dense(x, agg_pad, deg_pad, W_self, W_neigh,
                 bias.reshape(1, D), gamma.reshape(1, D), beta.reshape(1, D))
    return out


# speedup vs baseline: 1.1105x; 1.1105x over previous
import functools

import jax
import jax.numpy as jnp
from jax import lax
from jax.experimental import pallas as pl
from jax.experimental.pallas import tpu as pltpu
from jax.experimental.pallas import tpu_sc as plsc

N = 5000
D = 128
E = 32768
BATCH = 16

NC = 2
NS = 16
BPC = BATCH // NC

ROWS_PER_TILE = 320
NPAD = ROWS_PER_TILE * NS
EPT = E // NS
CHUNK = 128
NCHUNKS = EPT // CHUNK

_sc_mesh = plsc.VectorSubcoreMesh(core_axis_name="c", subcore_axis_name="s")


@functools.partial(
    pl.kernel,
    out_type=[
        jax.ShapeDtypeStruct((BATCH, NPAD, D), jnp.float32),
        jax.ShapeDtypeStruct((NPAD, D), jnp.float32),
    ],
    mesh=_sc_mesh,
    scratch_types=[
        pltpu.VMEM((NCHUNKS, CHUNK), jnp.int32),
        pltpu.VMEM((NCHUNKS, CHUNK), jnp.int32),
        pltpu.VMEM((4, CHUNK, D), jnp.float32),
        pltpu.VMEM((ROWS_PER_TILE // 5, D), jnp.float32),
        pltpu.VMEM_SHARED((NPAD, D), jnp.float32),
        pltpu.SemaphoreType.DMA,
        pltpu.SemaphoreType.DMA,
        pltpu.SemaphoreType.DMA,
        pltpu.SemaphoreType.DMA,
    ],
)
def _sc_scatter(x_hbm, src_hbm, dst_hbm, agg_hbm, deg_hbm,
                srcv, dstv, rows, zbuf, agg_sh, sem0, sem1, sem2, sem3):
    c = lax.axis_index("c")
    s = lax.axis_index("s")
    my = pl.ds(s * ROWS_PER_TILE, ROWS_PER_TILE)

    zero16 = jnp.zeros((16,), jnp.float32)

    def _zrow(i, _):
        for l in range(D // 16):
            zbuf[i, pl.ds(l * 16, 16)] = zero16
        return 0

    lax.fori_loop(0, ROWS_PER_TILE // 5, _zrow, 0)

    def _zero_my_slice():
        for z in range(5):
            pltpu.sync_copy(
                zbuf,
                agg_sh.at[pl.ds(s * ROWS_PER_TILE + z * (ROWS_PER_TILE // 5),
                                ROWS_PER_TILE // 5)])

    pltpu.sync_copy(src_hbm.at[pl.ds(s * NCHUNKS, NCHUNKS)], srcv)
    pltpu.sync_copy(dst_hbm.at[pl.ds(s * NCHUNKS, NCHUNKS)], dstv)

    @pl.when(c == 0)
    def _deg():
        one16 = jnp.full((16,), 1.0, jnp.float32)

        def _orow(i, _):
            for l in range(D // 16):
                rows[0, i, pl.ds(l * 16, 16)] = one16
            return 0

        lax.fori_loop(0, CHUNK, _orow, 0)
        _zero_my_slice()
        plsc.subcore_barrier()
        for j in range(NCHUNKS):
            pltpu.sync_copy(rows.at[0], agg_sh.at[dstv.at[j]], add=True)
        plsc.subcore_barrier()
        pltpu.sync_copy(agg_sh.at[my], deg_hbm.at[my])

    base0 = c * (BPC * N)

    def _shift(i, _):
        for l in range(CHUNK // 16):
            sl = pl.ds(l * 16, 16)
            srcv[i, sl] = srcv[i, sl] + base0
        return 0

    lax.fori_loop(0, NCHUNKS, _shift, 0)

    NBUF = 4
    sems = (sem0, sem1, sem2, sem3)

    def _issue_gather(j):
        return pltpu.async_copy(
            x_hbm.at[srcv.at[j]], rows.at[j % NBUF], sems[j % NBUF])

    _zero_my_slice()
    for j in range(NBUF - 1):
        _issue_gather(j)

    def _batch(b, _):
        plsc.subcore_barrier()
        scat = [None] * NCHUNKS
        gat = [None] * NCHUNKS
        for j in range(NCHUNKS):
            if j - 1 >= 0:
                scat[j - 1].wait()
            if j + NBUF - 1 < NCHUNKS:
                jn = j + NBUF - 1
                gat[jn] = _issue_gather(jn)
            if j < NBUF - 1:
                pltpu.make_async_copy(
                    x_hbm.at[srcv.at[j]], rows.at[j % NBUF],
                    sems[j % NBUF]).wait()
            else:
                gat[j].wait()
            scat[j] = pltpu.async_copy(
                rows.at[j % NBUF], agg_sh.at[dstv.at[j]], sems[j % NBUF],
                add=True)
        scat[NCHUNKS - 1].wait()
        plsc.subcore_barrier()

        def _bump(i, _):
            for l in range(CHUNK // 16):
                sl = pl.ds(l * 16, 16)
                srcv[i, sl] = srcv[i, sl] + N
            return 0

        lax.fori_loop(0, NCHUNKS, _bump, 0)

        @pl.when(b + 1 < BPC)
        def _prime_next():
            for j in range(NBUF - 1):
                _issue_gather(j)

        bg = c * BPC + b
        pltpu.sync_copy(agg_sh.at[my], agg_hbm.at[bg, my])
        _zero_my_slice()
        return 0

    lax.fori_loop(0, BPC, _batch, 0)


BN = 1000


def _dense_body(x_ref, agg_ref, deg_ref, ws_ref, wn_ref, b_ref, g_ref, be_ref,
                o_ref):
    xb = x_ref[0]
    inv = 1.0 / jnp.maximum(deg_ref[...], 1.0)
    neigh = agg_ref[0] * inv
    out = (jnp.dot(xb, ws_ref[...], preferred_element_type=jnp.float32)
           + jnp.dot(neigh, wn_ref[...], preferred_element_type=jnp.float32)
           + b_ref[...])
    mu = jnp.mean(out, axis=-1, keepdims=True)
    var = jnp.mean((out - mu) ** 2, axis=-1, keepdims=True)
    out = (out - mu) * lax.rsqrt(var + 1e-5) * g_ref[...] + be_ref[...]
    o_ref[0] = jnp.maximum(out, 0.0)


_dense = pl.pallas_call(
    _dense_body,
    grid=(N // BN, BATCH),
    in_specs=[
        pl.BlockSpec((1, BN, D), lambda j, b: (b, j, 0)),
        pl.BlockSpec((1, BN, D), lambda j, b: (b, j, 0)),
        pl.BlockSpec((BN, D), lambda j, b: (j, 0)),
        pl.BlockSpec((D, D), lambda j, b: (0, 0)),
        pl.BlockSpec((D, D), lambda j, b: (0, 0)),
        pl.BlockSpec((1, D), lambda j, b: (0, 0)),
        pl.BlockSpec((1, D), lambda j, b: (0, 0)),
        pl.BlockSpec((1, D), lambda j, b: (0, 0)),
    ],
    out_specs=pl.BlockSpec((1, BN, D), lambda j, b: (b, j, 0)),
    out_shape=jax.ShapeDtypeStruct((BATCH, N, D), jnp.float32),
    compiler_params=pltpu.CompilerParams(
        dimension_semantics=("parallel", "parallel")),
)


def kernel(x, edge_index, batch_size, W_self, W_neigh, bias, gamma, beta):
    x_flat = x.reshape(BATCH * N, D)
    src2d = edge_index[0].reshape(E // CHUNK, CHUNK)
    dst2d = edge_index[1].reshape(E // CHUNK, CHUNK)
    agg_pad, deg_pad = _sc_scatter(x_flat, src2d, dst2d)
    out = _dense(x, agg_pad, deg_pad, W_self, W_neigh,
                 bias.reshape(1, D), gamma.reshape(1, D), beta.reshape(1, D))
    return out


# speedup vs baseline: 1.2122x; 1.0916x over previous
import functools

import jax
import jax.numpy as jnp
from jax import lax
from jax.experimental import pallas as pl
from jax.experimental.pallas import tpu as pltpu
from jax.experimental.pallas import tpu_sc as plsc

N = 5000
D = 128
E = 32768
BATCH = 16
HALF = BATCH // 2

NC = 2
NS = 16

ROWS_PER_TILE = 320
NPAD = ROWS_PER_TILE * NS
EPT = E // NS
CHUNK = 128
NCHUNKS = EPT // CHUNK
NBUF = 4

_sc_mesh = plsc.VectorSubcoreMesh(core_axis_name="c", subcore_axis_name="s")

_SC_SCRATCH = [
    pltpu.VMEM((NCHUNKS, CHUNK), jnp.int32),
    pltpu.VMEM((NCHUNKS, CHUNK), jnp.int32),
    pltpu.VMEM((NBUF, CHUNK, D), jnp.float32),
    pltpu.VMEM((ROWS_PER_TILE // 5, D), jnp.float32),
    pltpu.VMEM_SHARED((NPAD, D), jnp.float32),
    pltpu.SemaphoreType.DMA,
    pltpu.SemaphoreType.DMA,
    pltpu.SemaphoreType.DMA,
    pltpu.SemaphoreType.DMA,
]


def _make_sc(nbatch, boff, with_deg):
    bpc = nbatch // NC
    if with_deg:
        out_types = [jax.ShapeDtypeStruct((nbatch, NPAD, D), jnp.float32),
                     jax.ShapeDtypeStruct((NPAD, D), jnp.float32)]
    else:
        out_types = jax.ShapeDtypeStruct((nbatch, NPAD, D), jnp.float32)

    @functools.partial(pl.kernel, out_type=out_types, mesh=_sc_mesh,
                       scratch_types=_SC_SCRATCH)
    def _sc(x_hbm, src_hbm, dst_hbm, *refs):
        if with_deg:
            (agg_hbm, deg_hbm, srcv, dstv, rows, zbuf, agg_sh,
             sem0, sem1, sem2, sem3) = refs
        else:
            (agg_hbm, srcv, dstv, rows, zbuf, agg_sh,
             sem0, sem1, sem2, sem3) = refs
        c = lax.axis_index("c")
        s = lax.axis_index("s")
        my = pl.ds(s * ROWS_PER_TILE, ROWS_PER_TILE)

        zero16 = jnp.zeros((16,), jnp.float32)

        def _zrow(i, _):
            for l in range(D // 16):
                zbuf[i, pl.ds(l * 16, 16)] = zero16
            return 0

        lax.fori_loop(0, ROWS_PER_TILE // 5, _zrow, 0)

        def _zero_my_slice():
            for z in range(5):
                pltpu.sync_copy(
                    zbuf,
                    agg_sh.at[pl.ds(
                        s * ROWS_PER_TILE + z * (ROWS_PER_TILE // 5),
                        ROWS_PER_TILE // 5)])

        pltpu.sync_copy(src_hbm.at[pl.ds(s * NCHUNKS, NCHUNKS)], srcv)
        pltpu.sync_copy(dst_hbm.at[pl.ds(s * NCHUNKS, NCHUNKS)], dstv)

        if with_deg:
            @pl.when(c == 0)
            def _deg():
                one16 = jnp.full((16,), 1.0, jnp.float32)

                def _orow(i, _):
                    for l in range(D // 16):
                        rows[0, i, pl.ds(l * 16, 16)] = one16
                    return 0

                lax.fori_loop(0, CHUNK, _orow, 0)
                _zero_my_slice()
                plsc.subcore_barrier()
                for j in range(NCHUNKS):
                    pltpu.sync_copy(rows.at[0], agg_sh.at[dstv.at[j]],
                                    add=True)
                plsc.subcore_barrier()
                pltpu.sync_copy(agg_sh.at[my], deg_hbm.at[my])

        base0 = (boff + c * bpc) * N

        def _shift(i, _):
            for l in range(CHUNK // 16):
                sl = pl.ds(l * 16, 16)
                srcv[i, sl] = srcv[i, sl] + base0
            return 0

        lax.fori_loop(0, NCHUNKS, _shift, 0)

        sems = (sem0, sem1, sem2, sem3)

        def _issue_gather(j):
            return pltpu.async_copy(
                x_hbm.at[srcv.at[j]], rows.at[j % NBUF], sems[j % NBUF])

        _zero_my_slice()
        for j in range(NBUF - 1):
            _issue_gather(j)

        def _batch(b, _):
            plsc.subcore_barrier()
            scat = [None] * NCHUNKS
            gat = [None] * NCHUNKS
            for j in range(NCHUNKS):
                if j - 1 >= 0:
                    scat[j - 1].wait()
                if j + NBUF - 1 < NCHUNKS:
                    jn = j + NBUF - 1
                    gat[jn] = _issue_gather(jn)
                if j < NBUF - 1:
                    pltpu.make_async_copy(
                        x_hbm.at[srcv.at[j]], rows.at[j % NBUF],
                        sems[j % NBUF]).wait()
                else:
                    gat[j].wait()
                scat[j] = pltpu.async_copy(
                    rows.at[j % NBUF], agg_sh.at[dstv.at[j]],
                    sems[j % NBUF], add=True)
            scat[NCHUNKS - 1].wait()
            plsc.subcore_barrier()

            def _bump(i, _):
                for l in range(CHUNK // 16):
                    sl = pl.ds(l * 16, 16)
                    srcv[i, sl] = srcv[i, sl] + N
                return 0

            lax.fori_loop(0, NCHUNKS, _bump, 0)

            @pl.when(b + 1 < bpc)
            def _prime_next():
                for j in range(NBUF - 1):
                    _issue_gather(j)

            bg = c * bpc + b
            pltpu.sync_copy(agg_sh.at[my], agg_hbm.at[bg, my])
            _zero_my_slice()
            return 0

        lax.fori_loop(0, bpc, _batch, 0)

    return _sc


_sc_first = _make_sc(HALF, 0, True)
_sc_second = _make_sc(HALF, HALF, False)


BN = 1000


def _dense_body(x_ref, agg_ref, deg_ref, ws_ref, wn_ref, b_ref, g_ref, be_ref,
                o_ref):
    xb = x_ref[0]
    inv = 1.0 / jnp.maximum(deg_ref[...], 1.0)
    neigh = agg_ref[0] * inv
    out = (jnp.dot(xb, ws_ref[...], preferred_element_type=jnp.float32)
           + jnp.dot(neigh, wn_ref[...], preferred_element_type=jnp.float32)
           + b_ref[...])
    mu = jnp.mean(out, axis=-1, keepdims=True)
    var = jnp.mean((out - mu) ** 2, axis=-1, keepdims=True)
    out = (out - mu) * lax.rsqrt(var + 1e-5) * g_ref[...] + be_ref[...]
    o_ref[0] = jnp.maximum(out, 0.0)


def _dense_body_aliased(x_ref, agg_ref, deg_ref, ws_ref, wn_ref, b_ref, g_ref,
                        be_ref, prev_ref, o_ref):
    del prev_ref
    _dense_body(x_ref, agg_ref, deg_ref, ws_ref, wn_ref, b_ref, g_ref, be_ref,
                o_ref)


def _make_dense(boff, aliased):
    in_specs = [
        pl.BlockSpec((1, BN, D), lambda j, b: (b + boff, j, 0)),
        pl.BlockSpec((1, BN, D), lambda j, b: (b, j, 0)),
        pl.BlockSpec((BN, D), lambda j, b: (j, 0)),
        pl.BlockSpec((D, D), lambda j, b: (0, 0)),
        pl.BlockSpec((D, D), lambda j, b: (0, 0)),
        pl.BlockSpec((1, D), lambda j, b: (0, 0)),
        pl.BlockSpec((1, D), lambda j, b: (0, 0)),
        pl.BlockSpec((1, D), lambda j, b: (0, 0)),
    ]
    body = _dense_body
    aliases = {}
    if aliased:
        in_specs.append(pl.BlockSpec(memory_space=pl.ANY))
        body = _dense_body_aliased
        aliases = {8: 0}
    return pl.pallas_call(
        body,
        grid=(N // BN, HALF),
        in_specs=in_specs,
        out_specs=pl.BlockSpec((1, BN, D), lambda j, b: (b + boff, j, 0)),
        out_shape=jax.ShapeDtypeStruct((BATCH, N, D), jnp.float32),
        input_output_aliases=aliases,
        compiler_params=pltpu.CompilerParams(
            dimension_semantics=("parallel", "parallel")),
    )


_dense_a = _make_dense(0, False)
_dense_b = _make_dense(HALF, True)


def kernel(x, edge_index, batch_size, W_self, W_neigh, bias, gamma, beta):
    x_flat = x.reshape(BATCH * N, D)
    src2d = edge_index[0].reshape(E // CHUNK, CHUNK)
    dst2d = edge_index[1].reshape(E // CHUNK, CHUNK)
    b2 = bias.reshape(1, D)
    g2 = gamma.reshape(1, D)
    be2 = beta.reshape(1, D)
    agg0, deg_pad = _sc_first(x_flat, src2d, dst2d)
    agg1 = _sc_second(x_flat, src2d, dst2d)
    half0 = _dense_a(x, agg0, deg_pad, W_self, W_neigh, b2, g2, be2)
    out = _dense_b(x, agg1, deg_pad, W_self, W_neigh, b2, g2, be2, half0)
    return out


# speedup vs baseline: 1.2160x; 1.0031x over previous
import functools

import jax
import jax.numpy as jnp
from jax import lax
from jax.experimental import pallas as pl
from jax.experimental.pallas import tpu as pltpu
from jax.experimental.pallas import tpu_sc as plsc

N = 5000
D = 128
E = 32768
BATCH = 16
HALF = BATCH // 2

NC = 2
NS = 16

ROWS_PER_TILE = 320
NPAD = ROWS_PER_TILE * NS
EPT = E // NS
CHUNK = 128
NCHUNKS = EPT // CHUNK
NBUF = 4

_sc_mesh = plsc.VectorSubcoreMesh(core_axis_name="c", subcore_axis_name="s")

_SC_SCRATCH = [
    pltpu.VMEM((NCHUNKS, CHUNK), jnp.int32),
    pltpu.VMEM((NCHUNKS, CHUNK), jnp.int32),
    pltpu.VMEM((NBUF, CHUNK, D), jnp.float32),
    pltpu.VMEM((ROWS_PER_TILE // 5, D), jnp.float32),
    pltpu.VMEM_SHARED((NPAD, D), jnp.float32),
    pltpu.SemaphoreType.DMA,
    pltpu.SemaphoreType.DMA,
    pltpu.SemaphoreType.DMA,
    pltpu.SemaphoreType.DMA,
]


def _make_sc(nbatch, boff, with_deg):
    bpc = nbatch // NC
    if with_deg:
        out_types = [jax.ShapeDtypeStruct((nbatch, NPAD, D), jnp.float32),
                     jax.ShapeDtypeStruct((NPAD, D), jnp.float32)]
    else:
        out_types = jax.ShapeDtypeStruct((nbatch, NPAD, D), jnp.float32)

    @functools.partial(pl.kernel, out_type=out_types, mesh=_sc_mesh,
                       scratch_types=_SC_SCRATCH)
    def _sc(x_hbm, src_hbm, dst_hbm, *refs):
        if with_deg:
            (agg_hbm, deg_hbm, srcv, dstv, rows, zbuf, agg_sh,
             sem0, sem1, sem2, sem3) = refs
        else:
            (agg_hbm, srcv, dstv, rows, zbuf, agg_sh,
             sem0, sem1, sem2, sem3) = refs
        c = lax.axis_index("c")
        s = lax.axis_index("s")
        my = pl.ds(s * ROWS_PER_TILE, ROWS_PER_TILE)

        zero16 = jnp.zeros((16,), jnp.float32)

        def _zrow(i, _):
            for l in range(D // 16):
                zbuf[i, pl.ds(l * 16, 16)] = zero16
            return 0

        lax.fori_loop(0, ROWS_PER_TILE // 5, _zrow, 0)

        def _zero_my_slice():
            for z in range(5):
                pltpu.sync_copy(
                    zbuf,
                    agg_sh.at[pl.ds(
                        s * ROWS_PER_TILE + z * (ROWS_PER_TILE // 5),
                        ROWS_PER_TILE // 5)])

        pltpu.sync_copy(src_hbm.at[pl.ds(s * NCHUNKS, NCHUNKS)], srcv)
        pltpu.sync_copy(dst_hbm.at[pl.ds(s * NCHUNKS, NCHUNKS)], dstv)

        if with_deg:
            @pl.when(c == 0)
            def _deg():
                one16 = jnp.full((16,), 1.0, jnp.float32)

                def _orow(i, _):
                    for l in range(D // 16):
                        rows[0, i, pl.ds(l * 16, 16)] = one16
                    return 0

                lax.fori_loop(0, CHUNK, _orow, 0)
                _zero_my_slice()
                plsc.subcore_barrier()
                for j in range(NCHUNKS):
                    pltpu.sync_copy(rows.at[0], agg_sh.at[dstv.at[j]],
                                    add=True)
                plsc.subcore_barrier()
                pltpu.sync_copy(agg_sh.at[my], deg_hbm.at[my])

        base0 = (boff + c * bpc) * N

        def _shift(i, _):
            for l in range(CHUNK // 16):
                sl = pl.ds(l * 16, 16)
                srcv[i, sl] = srcv[i, sl] + base0
            return 0

        lax.fori_loop(0, NCHUNKS, _shift, 0)

        sems = (sem0, sem1, sem2, sem3)

        def _issue_gather(j):
            return pltpu.async_copy(
                x_hbm.at[srcv.at[j]], rows.at[j % NBUF], sems[j % NBUF])

        _zero_my_slice()
        for j in range(NBUF - 1):
            _issue_gather(j)

        def _batch(b, _):
            plsc.subcore_barrier()
            scat = [None] * NCHUNKS
            gat = [None] * NCHUNKS
            for j in range(NCHUNKS):
                if j - 1 >= 0:
                    scat[j - 1].wait()
                if j + NBUF - 1 < NCHUNKS:
                    jn = j + NBUF - 1
                    gat[jn] = _issue_gather(jn)
                if j < NBUF - 1:
                    pltpu.make_async_copy(
                        x_hbm.at[srcv.at[j]], rows.at[j % NBUF],
                        sems[j % NBUF]).wait()
                else:
                    gat[j].wait()
                scat[j] = pltpu.async_copy(
                    rows.at[j % NBUF], agg_sh.at[dstv.at[j]],
                    sems[j % NBUF], add=True)
            scat[NCHUNKS - 1].wait()
            plsc.subcore_barrier()

            def _bump(i, _):
                for l in range(CHUNK // 16):
                    sl = pl.ds(l * 16, 16)
                    srcv[i, sl] = srcv[i, sl] + N
                return 0

            lax.fori_loop(0, NCHUNKS, _bump, 0)

            @pl.when(b + 1 < bpc)
            def _prime_next():
                for j in range(NBUF - 1):
                    _issue_gather(j)

            bg = c * bpc + b
            pltpu.sync_copy(agg_sh.at[my], agg_hbm.at[bg, my])
            _zero_my_slice()
            return 0

        lax.fori_loop(0, bpc, _batch, 0)

    return _sc


SPLIT = 12

_sc_first = _make_sc(SPLIT, 0, True)
_sc_second = _make_sc(BATCH - SPLIT, SPLIT, False)


BN = 1000


def _dense_body(x_ref, agg_ref, deg_ref, ws_ref, wn_ref, b_ref, g_ref, be_ref,
                o_ref):
    xb = x_ref[0]
    inv = 1.0 / jnp.maximum(deg_ref[...], 1.0)
    neigh = agg_ref[0] * inv
    out = (jnp.dot(xb, ws_ref[...], preferred_element_type=jnp.float32)
           + jnp.dot(neigh, wn_ref[...], preferred_element_type=jnp.float32)
           + b_ref[...])
    mu = jnp.mean(out, axis=-1, keepdims=True)
    var = jnp.mean((out - mu) ** 2, axis=-1, keepdims=True)
    out = (out - mu) * lax.rsqrt(var + 1e-5) * g_ref[...] + be_ref[...]
    o_ref[0] = jnp.maximum(out, 0.0)


def _dense_body_aliased(x_ref, agg_ref, deg_ref, ws_ref, wn_ref, b_ref, g_ref,
                        be_ref, prev_ref, o_ref):
    del prev_ref
    _dense_body(x_ref, agg_ref, deg_ref, ws_ref, wn_ref, b_ref, g_ref, be_ref,
                o_ref)


def _make_dense(boff, nb, aliased):
    in_specs = [
        pl.BlockSpec((1, BN, D), lambda j, b: (b + boff, j, 0)),
        pl.BlockSpec((1, BN, D), lambda j, b: (b, j, 0)),
        pl.BlockSpec((BN, D), lambda j, b: (j, 0)),
        pl.BlockSpec((D, D), lambda j, b: (0, 0)),
        pl.BlockSpec((D, D), lambda j, b: (0, 0)),
        pl.BlockSpec((1, D), lambda j, b: (0, 0)),
        pl.BlockSpec((1, D), lambda j, b: (0, 0)),
        pl.BlockSpec((1, D), lambda j, b: (0, 0)),
    ]
    body = _dense_body
    aliases = {}
    if aliased:
        in_specs.append(pl.BlockSpec(memory_space=pl.ANY))
        body = _dense_body_aliased
        aliases = {8: 0}
    return pl.pallas_call(
        body,
        grid=(N // BN, nb),
        in_specs=in_specs,
        out_specs=pl.BlockSpec((1, BN, D), lambda j, b: (b + boff, j, 0)),
        out_shape=jax.ShapeDtypeStruct((BATCH, N, D), jnp.float32),
        input_output_aliases=aliases,
        compiler_params=pltpu.CompilerParams(
            dimension_semantics=("parallel", "parallel")),
    )


_dense_a = _make_dense(0, SPLIT, False)
_dense_b = _make_dense(SPLIT, BATCH - SPLIT, True)


def kernel(x, edge_index, batch_size, W_self, W_neigh, bias, gamma, beta):
    x_flat = x.reshape(BATCH * N, D)
    src2d = edge_index[0].reshape(E // CHUNK, CHUNK)
    dst2d = edge_index[1].reshape(E // CHUNK, CHUNK)
    b2 = bias.reshape(1, D)
    g2 = gamma.reshape(1, D)
    be2 = beta.reshape(1, D)
    agg0, deg_pad = _sc_first(x_flat, src2d, dst2d)
    agg1 = _sc_second(x_flat, src2d, dst2d)
    half0 = _dense_a(x, agg0, deg_pad, W_self, W_neigh, b2, g2, be2)
    out = _dense_b(x, agg1, deg_pad, W_self, W_neigh, b2, g2, be2, half0)
    return out


# speedup vs baseline: 1.2179x; 1.0016x over previous
import functools

import jax
import jax.numpy as jnp
from jax import lax
from jax.experimental import pallas as pl
from jax.experimental.pallas import tpu as pltpu
from jax.experimental.pallas import tpu_sc as plsc

N = 5000
D = 128
E = 32768
BATCH = 16
HALF = BATCH // 2

NC = 2
NS = 16

ROWS_PER_TILE = 320
NPAD = ROWS_PER_TILE * NS
EPT = E // NS
CHUNK = 128
NCHUNKS = EPT // CHUNK
_sc_mesh = plsc.VectorSubcoreMesh(core_axis_name="c", subcore_axis_name="s")

DEGW = 16


def _sc_scratch(nbuf):
    return [
        pltpu.VMEM((NCHUNKS, CHUNK), jnp.int32),
        pltpu.VMEM((NCHUNKS, CHUNK), jnp.int32),
        pltpu.VMEM((nbuf, CHUNK, D), jnp.float32),
        pltpu.VMEM((ROWS_PER_TILE // 5, D), jnp.float32),
        pltpu.VMEM_SHARED((NPAD, D), jnp.float32),
        pltpu.SemaphoreType.DMA,
        pltpu.SemaphoreType.DMA,
        pltpu.SemaphoreType.DMA,
        pltpu.SemaphoreType.DMA,
    ]


def _make_sc(nbatch, boff, with_deg):
    bpc = nbatch // NC
    NBUF = 4
    if with_deg:
        out_types = [jax.ShapeDtypeStruct((nbatch, NPAD, D), jnp.float32),
                     jax.ShapeDtypeStruct((NPAD, D), jnp.float32)]
    else:
        out_types = jax.ShapeDtypeStruct((nbatch, NPAD, D), jnp.float32)
    scratch = _sc_scratch(NBUF)

    @functools.partial(pl.kernel, out_type=out_types, mesh=_sc_mesh,
                       scratch_types=scratch)
    def _sc(x_hbm, src_hbm, dst_hbm, *refs):
        if with_deg:
            (agg_hbm, deg_hbm, srcv, dstv, rows, zbuf, agg_sh,
             sem0, sem1, sem2, sem3) = refs
        else:
            (agg_hbm, srcv, dstv, rows, zbuf, agg_sh,
             sem0, sem1, sem2, sem3) = refs
        c = lax.axis_index("c")
        s = lax.axis_index("s")
        my = pl.ds(s * ROWS_PER_TILE, ROWS_PER_TILE)

        zero16 = jnp.zeros((16,), jnp.float32)

        def _zrow(i, _):
            for l in range(D // 16):
                zbuf[i, pl.ds(l * 16, 16)] = zero16
            return 0

        lax.fori_loop(0, ROWS_PER_TILE // 5, _zrow, 0)

        def _zero_my_slice():
            for z in range(5):
                pltpu.sync_copy(
                    zbuf,
                    agg_sh.at[pl.ds(
                        s * ROWS_PER_TILE + z * (ROWS_PER_TILE // 5),
                        ROWS_PER_TILE // 5)])

        pltpu.sync_copy(src_hbm.at[pl.ds(s * NCHUNKS, NCHUNKS)], srcv)
        pltpu.sync_copy(dst_hbm.at[pl.ds(s * NCHUNKS, NCHUNKS)], dstv)

        if with_deg:
            @pl.when(c == 0)
            def _deg():
                one16 = jnp.full((16,), 1.0, jnp.float32)

                def _orow(i, _):
                    for l in range(D // 16):
                        rows[0, i, pl.ds(l * 16, 16)] = one16
                    return 0

                lax.fori_loop(0, CHUNK, _orow, 0)
                _zero_my_slice()
                plsc.subcore_barrier()
                dsc = [None] * NCHUNKS
                for j in range(NCHUNKS):
                    if j - 2 >= 0:
                        dsc[j - 2].wait()
                    dsc[j] = pltpu.async_copy(
                        rows.at[0], agg_sh.at[dstv.at[j]],
                        (sem0, sem1)[j % 2], add=True)
                dsc[NCHUNKS - 2].wait()
                dsc[NCHUNKS - 1].wait()
                plsc.subcore_barrier()
                pltpu.sync_copy(agg_sh.at[my], deg_hbm.at[my])

        base0 = (boff + c * bpc) * N

        def _shift(i, _):
            for l in range(CHUNK // 16):
                sl = pl.ds(l * 16, 16)
                srcv[i, sl] = srcv[i, sl] + base0
            return 0

        lax.fori_loop(0, NCHUNKS, _shift, 0)

        sems = (sem0, sem1, sem2, sem3)

        def _issue_gather(j):
            return pltpu.async_copy(
                x_hbm.at[srcv.at[j]], rows.at[j % NBUF], sems[j % NBUF])

        _zero_my_slice()
        for j in range(NBUF - 1):
            _issue_gather(j)

        def _batch(b, _):
            plsc.subcore_barrier()
            scat = [None] * NCHUNKS
            gat = [None] * NCHUNKS
            for j in range(NCHUNKS):
                if j - 1 >= 0:
                    scat[j - 1].wait()
                if j + NBUF - 1 < NCHUNKS:
                    jn = j + NBUF - 1
                    gat[jn] = _issue_gather(jn)
                if j < NBUF - 1:
                    pltpu.make_async_copy(
                        x_hbm.at[srcv.at[j]], rows.at[j % NBUF],
                        sems[j % NBUF]).wait()
                else:
                    gat[j].wait()
                scat[j] = pltpu.async_copy(
                    rows.at[j % NBUF], agg_sh.at[dstv.at[j]],
                    sems[j % NBUF], add=True)
            scat[NCHUNKS - 1].wait()
            plsc.subcore_barrier()

            def _bump(i, _):
                for l in range(CHUNK // 16):
                    sl = pl.ds(l * 16, 16)
                    srcv[i, sl] = srcv[i, sl] + N
                return 0

            lax.fori_loop(0, NCHUNKS, _bump, 0)

            @pl.when(b + 1 < bpc)
            def _prime_next():
                for j in range(NBUF - 1):
                    _issue_gather(j)

            bg = c * bpc + b
            pltpu.sync_copy(agg_sh.at[my], agg_hbm.at[bg, my])
            _zero_my_slice()
            return 0

        lax.fori_loop(0, bpc, _batch, 0)

    return _sc


SPLIT = 12

_sc_first = _make_sc(SPLIT, 0, True)
_sc_second = _make_sc(BATCH - SPLIT, SPLIT, False)


BN = 1000


def _dense_body(x_ref, agg_ref, deg_ref, ws_ref, wn_ref, b_ref, g_ref, be_ref,
                o_ref):
    xb = x_ref[0]
    inv = 1.0 / jnp.maximum(deg_ref[...], 1.0)
    neigh = agg_ref[0] * inv
    out = (jnp.dot(xb, ws_ref[...], preferred_element_type=jnp.float32)
           + jnp.dot(neigh, wn_ref[...], preferred_element_type=jnp.float32)
           + b_ref[...])
    mu = jnp.mean(out, axis=-1, keepdims=True)
    var = jnp.mean((out - mu) ** 2, axis=-1, keepdims=True)
    out = (out - mu) * lax.rsqrt(var + 1e-5) * g_ref[...] + be_ref[...]
    o_ref[0] = jnp.maximum(out, 0.0)


def _dense_body_aliased(x_ref, agg_ref, deg_ref, ws_ref, wn_ref, b_ref, g_ref,
                        be_ref, prev_ref, o_ref):
    del prev_ref
    _dense_body(x_ref, agg_ref, deg_ref, ws_ref, wn_ref, b_ref, g_ref, be_ref,
                o_ref)


def _make_dense(boff, nb, aliased):
    in_specs = [
        pl.BlockSpec((1, BN, D), lambda j, b: (b + boff, j, 0)),
        pl.BlockSpec((1, BN, D), lambda j, b: (b, j, 0)),
        pl.BlockSpec((BN, D), lambda j, b: (j, 0)),
        pl.BlockSpec((D, D), lambda j, b: (0, 0)),
        pl.BlockSpec((D, D), lambda j, b: (0, 0)),
        pl.BlockSpec((1, D), lambda j, b: (0, 0)),
        pl.BlockSpec((1, D), lambda j, b: (0, 0)),
        pl.BlockSpec((1, D), lambda j, b: (0, 0)),
    ]
    body = _dense_body
    aliases = {}
    if aliased:
        in_specs.append(pl.BlockSpec(memory_space=pl.ANY))
        body = _dense_body_aliased
        aliases = {8: 0}
    return pl.pallas_call(
        body,
        grid=(N // BN, nb),
        in_specs=in_specs,
        out_specs=pl.BlockSpec((1, BN, D), lambda j, b: (b + boff, j, 0)),
        out_shape=jax.ShapeDtypeStruct((BATCH, N, D), jnp.float32),
        input_output_aliases=aliases,
        compiler_params=pltpu.CompilerParams(
            dimension_semantics=("parallel", "parallel")),
    )


_dense_a = _make_dense(0, SPLIT, False)
_dense_b = _make_dense(SPLIT, BATCH - SPLIT, True)


def kernel(x, edge_index, batch_size, W_self, W_neigh, bias, gamma, beta):
    x_flat = x.reshape(BATCH * N, D)
    src2d = edge_index[0].reshape(E // CHUNK, CHUNK)
    dst2d = edge_index[1].reshape(E // CHUNK, CHUNK)
    b2 = bias.reshape(1, D)
    g2 = gamma.reshape(1, D)
    be2 = beta.reshape(1, D)
    agg0, deg_pad = _sc_first(x_flat, src2d, dst2d)
    agg1 = _sc_second(x_flat, src2d, dst2d)
    half0 = _dense_a(x, agg0, deg_pad, W_self, W_neigh, b2, g2, be2)
    out = _dense_b(x, agg1, deg_pad, W_self, W_neigh, b2, g2, be2, half0)
    return out


# speedup vs baseline: 1.3268x; 1.0894x over previous
import functools

import jax
import jax.numpy as jnp
from jax import lax
from jax.experimental import pallas as pl
from jax.experimental.pallas import tpu as pltpu
from jax.experimental.pallas import tpu_sc as plsc

N = 5000
D = 128
E = 32768
BATCH = 16
HALF = BATCH // 2

NC = 2
NS = 16

ROWS_PER_TILE = 320
NPAD = ROWS_PER_TILE * NS
EPT = E // NS
CHUNK = 128
NCHUNKS = EPT // CHUNK
_sc_mesh = plsc.VectorSubcoreMesh(core_axis_name="c", subcore_axis_name="s")

DEGW = 16


def _sc_scratch(nbuf):
    return [
        pltpu.VMEM((NCHUNKS, CHUNK), jnp.int32),
        pltpu.VMEM((NCHUNKS, CHUNK), jnp.int32),
        pltpu.VMEM((nbuf, CHUNK, D), jnp.float32),
        pltpu.VMEM((ROWS_PER_TILE // 5, D), jnp.float32),
        pltpu.VMEM_SHARED((NPAD, D), jnp.float32),
        pltpu.SemaphoreType.DMA,
        pltpu.SemaphoreType.DMA,
        pltpu.SemaphoreType.DMA,
        pltpu.SemaphoreType.DMA,
    ]


def _make_sc(nbatch, boff, with_deg):
    bpc = nbatch // NC
    NBUF = 4
    if with_deg:
        out_types = [jax.ShapeDtypeStruct((nbatch, NPAD, D), jnp.float32),
                     jax.ShapeDtypeStruct((NPAD, D), jnp.float32)]
    else:
        out_types = jax.ShapeDtypeStruct((nbatch, NPAD, D), jnp.float32)
    scratch = _sc_scratch(NBUF)

    @functools.partial(pl.kernel, out_type=out_types, mesh=_sc_mesh,
                       scratch_types=scratch)
    def _sc(x_hbm, src_hbm, dst_hbm, *refs):
        if with_deg:
            (agg_hbm, deg_hbm, srcv, dstv, rows, zbuf, agg_sh,
             sem0, sem1, sem2, sem3) = refs
        else:
            (agg_hbm, srcv, dstv, rows, zbuf, agg_sh,
             sem0, sem1, sem2, sem3) = refs
        c = lax.axis_index("c")
        s = lax.axis_index("s")
        my = pl.ds(s * ROWS_PER_TILE, ROWS_PER_TILE)

        zero16 = jnp.zeros((16,), jnp.float32)

        def _zrow(i, _):
            for l in range(D // 16):
                zbuf[i, pl.ds(l * 16, 16)] = zero16
            return 0

        lax.fori_loop(0, ROWS_PER_TILE // 5, _zrow, 0)

        def _zero_my_slice():
            for z in range(5):
                pltpu.sync_copy(
                    zbuf,
                    agg_sh.at[pl.ds(
                        s * ROWS_PER_TILE + z * (ROWS_PER_TILE // 5),
                        ROWS_PER_TILE // 5)])

        pltpu.sync_copy(src_hbm.at[pl.ds(s * NCHUNKS, NCHUNKS)], srcv)
        pltpu.sync_copy(dst_hbm.at[pl.ds(s * NCHUNKS, NCHUNKS)], dstv)

        if with_deg:
            @pl.when(c == 0)
            def _deg():
                one16 = jnp.full((16,), 1.0, jnp.float32)

                def _orow(i, _):
                    for l in range(D // 16):
                        rows[0, i, pl.ds(l * 16, 16)] = one16
                    return 0

                lax.fori_loop(0, CHUNK, _orow, 0)
                _zero_my_slice()
                plsc.subcore_barrier()
                dsc = [None] * NCHUNKS
                for j in range(NCHUNKS):
                    if j - 2 >= 0:
                        dsc[j - 2].wait()
                    dsc[j] = pltpu.async_copy(
                        rows.at[0], agg_sh.at[dstv.at[j]],
                        (sem0, sem1)[j % 2], add=True)
                dsc[NCHUNKS - 2].wait()
                dsc[NCHUNKS - 1].wait()
                plsc.subcore_barrier()
                pltpu.sync_copy(agg_sh.at[my], deg_hbm.at[my])

        base0 = (boff + c * bpc) * N

        def _shift(i, _):
            for l in range(CHUNK // 16):
                sl = pl.ds(l * 16, 16)
                srcv[i, sl] = srcv[i, sl] + base0
            return 0

        lax.fori_loop(0, NCHUNKS, _shift, 0)

        sems = (sem0, sem1, sem2, sem3)

        def _issue_gather(j):
            return pltpu.async_copy(
                x_hbm.at[srcv.at[j]], rows.at[j % NBUF], sems[j % NBUF])

        _zero_my_slice()
        for j in range(NBUF - 1):
            _issue_gather(j)

        def _batch(b, _):
            plsc.subcore_barrier()
            scat = [None] * NCHUNKS
            gat = [None] * NCHUNKS
            for j in range(NCHUNKS):
                if j - 1 >= 0:
                    scat[j - 1].wait()
                if j + NBUF - 1 < NCHUNKS:
                    jn = j + NBUF - 1
                    gat[jn] = _issue_gather(jn)
                if j < NBUF - 1:
                    pltpu.make_async_copy(
                        x_hbm.at[srcv.at[j]], rows.at[j % NBUF],
                        sems[j % NBUF]).wait()
                else:
                    gat[j].wait()
                scat[j] = pltpu.async_copy(
                    rows.at[j % NBUF], agg_sh.at[dstv.at[j]],
                    sems[j % NBUF], add=True)
            scat[NCHUNKS - 1].wait()
            plsc.subcore_barrier()

            def _bump(i, _):
                for l in range(CHUNK // 16):
                    sl = pl.ds(l * 16, 16)
                    srcv[i, sl] = srcv[i, sl] + N
                return 0

            lax.fori_loop(0, NCHUNKS, _bump, 0)

            @pl.when(b + 1 < bpc)
            def _prime_next():
                for j in range(NBUF - 1):
                    _issue_gather(j)

            bg = c * bpc + b
            pltpu.sync_copy(agg_sh.at[my], agg_hbm.at[bg, my])
            _zero_my_slice()
            return 0

        lax.fori_loop(0, bpc, _batch, 0)

    return _sc


SPLIT = 12

_sc_first = _make_sc(SPLIT, 0, True)
_sc_second = _make_sc(BATCH - SPLIT, SPLIT, False)


BN = 5000


def _dense_body(x_ref, agg_ref, deg_ref, ws_ref, wn_ref, b_ref, g_ref, be_ref,
                o_ref):
    xb = x_ref[0]
    inv = 1.0 / jnp.maximum(deg_ref[...], 1.0)
    neigh = agg_ref[0] * inv
    out = (jnp.dot(xb, ws_ref[...], preferred_element_type=jnp.float32)
           + jnp.dot(neigh, wn_ref[...], preferred_element_type=jnp.float32)
           + b_ref[...])
    mu = jnp.mean(out, axis=-1, keepdims=True)
    var = jnp.mean((out - mu) ** 2, axis=-1, keepdims=True)
    out = (out - mu) * lax.rsqrt(var + 1e-5) * g_ref[...] + be_ref[...]
    o_ref[0] = jnp.maximum(out, 0.0)


def _dense_body_aliased(x_ref, agg_ref, deg_ref, ws_ref, wn_ref, b_ref, g_ref,
                        be_ref, prev_ref, o_ref):
    del prev_ref
    _dense_body(x_ref, agg_ref, deg_ref, ws_ref, wn_ref, b_ref, g_ref, be_ref,
                o_ref)


def _make_dense(boff, nb, aliased):
    in_specs = [
        pl.BlockSpec((1, BN, D), lambda j, b: (b + boff, j, 0)),
        pl.BlockSpec((1, BN, D), lambda j, b: (b, j, 0)),
        pl.BlockSpec((BN, D), lambda j, b: (j, 0)),
        pl.BlockSpec((D, D), lambda j, b: (0, 0)),
        pl.BlockSpec((D, D), lambda j, b: (0, 0)),
        pl.BlockSpec((1, D), lambda j, b: (0, 0)),
        pl.BlockSpec((1, D), lambda j, b: (0, 0)),
        pl.BlockSpec((1, D), lambda j, b: (0, 0)),
    ]
    body = _dense_body
    aliases = {}
    if aliased:
        in_specs.append(pl.BlockSpec(memory_space=pl.ANY))
        body = _dense_body_aliased
        aliases = {8: 0}
    return pl.pallas_call(
        body,
        grid=(N // BN, nb),
        in_specs=in_specs,
        out_specs=pl.BlockSpec((1, BN, D), lambda j, b: (b + boff, j, 0)),
        out_shape=jax.ShapeDtypeStruct((BATCH, N, D), jnp.float32),
        input_output_aliases=aliases,
        compiler_params=pltpu.CompilerParams(
            dimension_semantics=("parallel", "parallel")),
    )


_dense_a = _make_dense(0, SPLIT, False)
_dense_b = _make_dense(SPLIT, BATCH - SPLIT, True)


def kernel(x, edge_index, batch_size, W_self, W_neigh, bias, gamma, beta):
    x_flat = x.reshape(BATCH * N, D)
    src2d = edge_index[0].reshape(E // CHUNK, CHUNK)
    dst2d = edge_index[1].reshape(E // CHUNK, CHUNK)
    b2 = bias.reshape(1, D)
    g2 = gamma.reshape(1, D)
    be2 = beta.reshape(1, D)
    agg0, deg_pad = _sc_first(x_flat, src2d, dst2d)
    agg1 = _sc_second(x_flat, src2d, dst2d)
    half0 = _dense_a(x, agg0, deg_pad, W_self, W_neigh, b2, g2, be2)
    out = _dense_b(x, agg1, deg_pad, W_self, W_neigh, b2, g2, be2, half0)
    return out


# speedup vs baseline: 1.3376x; 1.0081x over previous
import functools

import jax
import jax.numpy as jnp
from jax import lax
from jax.experimental import pallas as pl
from jax.experimental.pallas import tpu as pltpu
from jax.experimental.pallas import tpu_sc as plsc

N = 5000
D = 128
E = 32768
BATCH = 16
HALF = BATCH // 2

NC = 2
NS = 16

ROWS_PER_TILE = 320
NPAD = ROWS_PER_TILE * NS
EPT = E // NS
CHUNK = 128
NCHUNKS = EPT // CHUNK
_sc_mesh = plsc.VectorSubcoreMesh(core_axis_name="c", subcore_axis_name="s")

DEGW = 16


def _sc_scratch(nbuf):
    return [
        pltpu.VMEM((NCHUNKS, CHUNK), jnp.int32),
        pltpu.VMEM((NCHUNKS, CHUNK), jnp.int32),
        pltpu.VMEM((nbuf, CHUNK, D), jnp.float32),
        pltpu.VMEM((ROWS_PER_TILE // 5, D), jnp.float32),
        pltpu.VMEM_SHARED((NPAD, D), jnp.float32),
        pltpu.SemaphoreType.DMA,
        pltpu.SemaphoreType.DMA,
        pltpu.SemaphoreType.DMA,
        pltpu.SemaphoreType.DMA,
    ]


def _make_sc(nbatch, boff, with_deg):
    bpc = nbatch // NC
    NBUF = 4
    if with_deg:
        out_types = [jax.ShapeDtypeStruct((nbatch, NPAD, D), jnp.float32),
                     jax.ShapeDtypeStruct((NC, NPAD, D), jnp.float32)]
    else:
        out_types = jax.ShapeDtypeStruct((nbatch, NPAD, D), jnp.float32)
    scratch = _sc_scratch(NBUF)

    @functools.partial(pl.kernel, out_type=out_types, mesh=_sc_mesh,
                       scratch_types=scratch)
    def _sc(x_hbm, src_hbm, dst_hbm, *refs):
        if with_deg:
            (agg_hbm, deg_hbm, srcv, dstv, rows, zbuf, agg_sh,
             sem0, sem1, sem2, sem3) = refs
        else:
            (agg_hbm, srcv, dstv, rows, zbuf, agg_sh,
             sem0, sem1, sem2, sem3) = refs
        c = lax.axis_index("c")
        s = lax.axis_index("s")
        my = pl.ds(s * ROWS_PER_TILE, ROWS_PER_TILE)

        zero16 = jnp.zeros((16,), jnp.float32)

        def _zrow(i, _):
            for l in range(D // 16):
                zbuf[i, pl.ds(l * 16, 16)] = zero16
            return 0

        lax.fori_loop(0, ROWS_PER_TILE // 5, _zrow, 0)

        def _zero_my_slice():
            for z in range(5):
                pltpu.sync_copy(
                    zbuf,
                    agg_sh.at[pl.ds(
                        s * ROWS_PER_TILE + z * (ROWS_PER_TILE // 5),
                        ROWS_PER_TILE // 5)])

        pltpu.sync_copy(src_hbm.at[pl.ds(s * NCHUNKS, NCHUNKS)], srcv)
        pltpu.sync_copy(dst_hbm.at[pl.ds(s * NCHUNKS, NCHUNKS)], dstv)

        if with_deg:
            one16 = jnp.full((16,), 1.0, jnp.float32)

            def _orow(i, _):
                for l in range(D // 16):
                    rows[0, i, pl.ds(l * 16, 16)] = one16
                return 0

            lax.fori_loop(0, CHUNK, _orow, 0)
            _zero_my_slice()
            plsc.subcore_barrier()
            half_chunks = NCHUNKS // 2
            dsc = [None] * half_chunks
            for jj in range(half_chunks):
                if jj - 2 >= 0:
                    dsc[jj - 2].wait()
                dsc[jj] = pltpu.async_copy(
                    rows.at[0], agg_sh.at[dstv.at[c * half_chunks + jj]],
                    (sem0, sem1)[jj % 2], add=True)
            dsc[half_chunks - 2].wait()
            dsc[half_chunks - 1].wait()
            plsc.subcore_barrier()
            pltpu.sync_copy(agg_sh.at[my], deg_hbm.at[c, my])

        base0 = (boff + c * bpc) * N

        def _shift(i, _):
            for l in range(CHUNK // 16):
                sl = pl.ds(l * 16, 16)
                srcv[i, sl] = srcv[i, sl] + base0
            return 0

        lax.fori_loop(0, NCHUNKS, _shift, 0)

        sems = (sem0, sem1, sem2, sem3)

        def _issue_gather(j):
            return pltpu.async_copy(
                x_hbm.at[srcv.at[j]], rows.at[j % NBUF], sems[j % NBUF])

        _zero_my_slice()
        for j in range(NBUF - 1):
            _issue_gather(j)

        def _batch(b, _):
            plsc.subcore_barrier()
            scat = [None] * NCHUNKS
            gat = [None] * NCHUNKS
            for j in range(NCHUNKS):
                if j - 1 >= 0:
                    scat[j - 1].wait()
                if j + NBUF - 1 < NCHUNKS:
                    jn = j + NBUF - 1
                    gat[jn] = _issue_gather(jn)
                if j < NBUF - 1:
                    pltpu.make_async_copy(
                        x_hbm.at[srcv.at[j]], rows.at[j % NBUF],
                        sems[j % NBUF]).wait()
                else:
                    gat[j].wait()
                scat[j] = pltpu.async_copy(
                    rows.at[j % NBUF], agg_sh.at[dstv.at[j]],
                    sems[j % NBUF], add=True)
            scat[NCHUNKS - 1].wait()
            plsc.subcore_barrier()

            def _bump(i, _):
                for l in range(CHUNK // 16):
                    sl = pl.ds(l * 16, 16)
                    srcv[i, sl] = srcv[i, sl] + N
                return 0

            lax.fori_loop(0, NCHUNKS, _bump, 0)

            @pl.when(b + 1 < bpc)
            def _prime_next():
                for j in range(NBUF - 1):
                    _issue_gather(j)

            bg = c * bpc + b
            pltpu.sync_copy(agg_sh.at[my], agg_hbm.at[bg, my])
            _zero_my_slice()
            return 0

        lax.fori_loop(0, bpc, _batch, 0)

    return _sc


SPLIT = 12

_sc_first = _make_sc(SPLIT, 0, True)
_sc_second = _make_sc(BATCH - SPLIT, SPLIT, False)


BN = 5000


def _dense_body(x_ref, agg_ref, deg_ref, ws_ref, wn_ref, b_ref, g_ref, be_ref,
                o_ref):
    xb = x_ref[0]
    inv = 1.0 / jnp.maximum(deg_ref[0] + deg_ref[1], 1.0)
    neigh = agg_ref[0] * inv
    out = (jnp.dot(xb, ws_ref[...], preferred_element_type=jnp.float32)
           + jnp.dot(neigh, wn_ref[...], preferred_element_type=jnp.float32)
           + b_ref[...])
    mu = jnp.mean(out, axis=-1, keepdims=True)
    var = jnp.mean((out - mu) ** 2, axis=-1, keepdims=True)
    out = (out - mu) * lax.rsqrt(var + 1e-5) * g_ref[...] + be_ref[...]
    o_ref[0] = jnp.maximum(out, 0.0)


def _dense_body_aliased(x_ref, agg_ref, deg_ref, ws_ref, wn_ref, b_ref, g_ref,
                        be_ref, prev_ref, o_ref):
    del prev_ref
    _dense_body(x_ref, agg_ref, deg_ref, ws_ref, wn_ref, b_ref, g_ref, be_ref,
                o_ref)


def _make_dense(boff, nb, aliased):
    in_specs = [
        pl.BlockSpec((1, BN, D), lambda j, b: (b + boff, j, 0)),
        pl.BlockSpec((1, BN, D), lambda j, b: (b, j, 0)),
        pl.BlockSpec((NC, BN, D), lambda j, b: (0, j, 0)),
        pl.BlockSpec((D, D), lambda j, b: (0, 0)),
        pl.BlockSpec((D, D), lambda j, b: (0, 0)),
        pl.BlockSpec((1, D), lambda j, b: (0, 0)),
        pl.BlockSpec((1, D), lambda j, b: (0, 0)),
        pl.BlockSpec((1, D), lambda j, b: (0, 0)),
    ]
    body = _dense_body
    aliases = {}
    if aliased:
        in_specs.append(pl.BlockSpec(memory_space=pl.ANY))
        body = _dense_body_aliased
        aliases = {8: 0}
    return pl.pallas_call(
        body,
        grid=(N // BN, nb),
        in_specs=in_specs,
        out_specs=pl.BlockSpec((1, BN, D), lambda j, b: (b + boff, j, 0)),
        out_shape=jax.ShapeDtypeStruct((BATCH, N, D), jnp.float32),
        input_output_aliases=aliases,
        compiler_params=pltpu.CompilerParams(
            dimension_semantics=("parallel", "parallel")),
    )


_dense_a = _make_dense(0, SPLIT, False)
_dense_b = _make_dense(SPLIT, BATCH - SPLIT, True)


def kernel(x, edge_index, batch_size, W_self, W_neigh, bias, gamma, beta):
    x_flat = x.reshape(BATCH * N, D)
    src2d = edge_index[0].reshape(E // CHUNK, CHUNK)
    dst2d = edge_index[1].reshape(E // CHUNK, CHUNK)
    b2 = bias.reshape(1, D)
    g2 = gamma.reshape(1, D)
    be2 = beta.reshape(1, D)
    agg0, deg_pad = _sc_first(x_flat, src2d, dst2d)
    agg1 = _sc_second(x_flat, src2d, dst2d)
    half0 = _dense_a(x, agg0, deg_pad, W_self, W_neigh, b2, g2, be2)
    out = _dense_b(x, agg1, deg_pad, W_self, W_neigh, b2, g2, be2, half0)
    return out


# speedup vs baseline: 1.3387x; 1.0008x over previous
import functools

import jax
import jax.numpy as jnp
from jax import lax
from jax.experimental import pallas as pl
from jax.experimental.pallas import tpu as pltpu
from jax.experimental.pallas import tpu_sc as plsc

N = 5000
D = 128
E = 32768
BATCH = 16

NC = 2
NS = 16

ROWS_PER_TILE = 320
NPAD = ROWS_PER_TILE * NS
EPT = E // NS
CHUNK = 128
NCHUNKS = EPT // CHUNK
_sc_mesh = plsc.VectorSubcoreMesh(core_axis_name="c", subcore_axis_name="s")


def _sc_scratch(nbuf):
    return [
        pltpu.VMEM((NCHUNKS, CHUNK), jnp.int32),
        pltpu.VMEM((NCHUNKS, CHUNK), jnp.int32),
        pltpu.VMEM((nbuf, CHUNK, D), jnp.float32),
        pltpu.VMEM((ROWS_PER_TILE // 5, D), jnp.float32),
        pltpu.VMEM_SHARED((NPAD, D), jnp.float32),
        pltpu.SemaphoreType.DMA,
        pltpu.SemaphoreType.DMA,
        pltpu.SemaphoreType.DMA,
        pltpu.SemaphoreType.DMA,
    ]


def _make_sc(nbatch, boff, with_deg):
    bpc = nbatch // NC
    NBUF = 4
    if with_deg:
        out_types = [jax.ShapeDtypeStruct((nbatch, NPAD, D), jnp.float32),
                     jax.ShapeDtypeStruct((NC, NPAD, D), jnp.float32)]
    else:
        out_types = jax.ShapeDtypeStruct((nbatch, NPAD, D), jnp.float32)
    scratch = _sc_scratch(NBUF)

    @functools.partial(pl.kernel, out_type=out_types, mesh=_sc_mesh,
                       scratch_types=scratch)
    def _sc(x_hbm, src_hbm, dst_hbm, *refs):
        if with_deg:
            (agg_hbm, deg_hbm, srcv, dstv, rows, zbuf, agg_sh,
             sem0, sem1, sem2, sem3) = refs
        else:
            (agg_hbm, srcv, dstv, rows, zbuf, agg_sh,
             sem0, sem1, sem2, sem3) = refs
        c = lax.axis_index("c")
        s = lax.axis_index("s")
        my = pl.ds(s * ROWS_PER_TILE, ROWS_PER_TILE)

        zero16 = jnp.zeros((16,), jnp.float32)

        def _zrow(i, _):
            for l in range(D // 16):
                zbuf[i, pl.ds(l * 16, 16)] = zero16
            return 0

        lax.fori_loop(0, ROWS_PER_TILE // 5, _zrow, 0)

        def _zero_my_slice():
            for z in range(5):
                pltpu.sync_copy(
                    zbuf,
                    agg_sh.at[pl.ds(
                        s * ROWS_PER_TILE + z * (ROWS_PER_TILE // 5),
                        ROWS_PER_TILE // 5)])

        pltpu.sync_copy(src_hbm.at[pl.ds(s * NCHUNKS, NCHUNKS)], srcv)
        pltpu.sync_copy(dst_hbm.at[pl.ds(s * NCHUNKS, NCHUNKS)], dstv)

        if with_deg:
            one16 = jnp.full((16,), 1.0, jnp.float32)

            def _orow(i, _):
                for l in range(D // 16):
                    rows[0, i, pl.ds(l * 16, 16)] = one16
                return 0

            lax.fori_loop(0, CHUNK, _orow, 0)
            _zero_my_slice()
            plsc.subcore_barrier()
            half_chunks = NCHUNKS // 2
            dsc = [None] * half_chunks
            for jj in range(half_chunks):
                if jj - 2 >= 0:
                    dsc[jj - 2].wait()
                dsc[jj] = pltpu.async_copy(
                    rows.at[0], agg_sh.at[dstv.at[c * half_chunks + jj]],
                    (sem0, sem1)[jj % 2], add=True)
            dsc[half_chunks - 2].wait()
            dsc[half_chunks - 1].wait()
            plsc.subcore_barrier()
            pltpu.sync_copy(agg_sh.at[my], deg_hbm.at[c, my])

        base0 = (boff + c * bpc) * N

        def _shift(i, _):
            for l in range(CHUNK // 16):
                sl = pl.ds(l * 16, 16)
                srcv[i, sl] = srcv[i, sl] + base0
            return 0

        lax.fori_loop(0, NCHUNKS, _shift, 0)

        sems = (sem0, sem1, sem2, sem3)

        def _issue_gather(j):
            return pltpu.async_copy(
                x_hbm.at[srcv.at[j]], rows.at[j % NBUF], sems[j % NBUF])

        _zero_my_slice()
        for j in range(NBUF - 1):
            _issue_gather(j)

        def _batch(b, _):
            plsc.subcore_barrier()
            scat = [None] * NCHUNKS
            gat = [None] * NCHUNKS
            for j in range(NCHUNKS):
                if j - 1 >= 0:
                    scat[j - 1].wait()
                if j + NBUF - 1 < NCHUNKS:
                    jn = j + NBUF - 1
                    gat[jn] = _issue_gather(jn)
                if j < NBUF - 1:
                    pltpu.make_async_copy(
                        x_hbm.at[srcv.at[j]], rows.at[j % NBUF],
                        sems[j % NBUF]).wait()
                else:
                    gat[j].wait()
                scat[j] = pltpu.async_copy(
                    rows.at[j % NBUF], agg_sh.at[dstv.at[j]],
                    sems[j % NBUF], add=True)
            scat[NCHUNKS - 1].wait()
            plsc.subcore_barrier()

            def _bump(i, _):
                for l in range(CHUNK // 16):
                    sl = pl.ds(l * 16, 16)
                    srcv[i, sl] = srcv[i, sl] + N
                return 0

            lax.fori_loop(0, NCHUNKS, _bump, 0)

            @pl.when(b + 1 < bpc)
            def _prime_next():
                for j in range(NBUF - 1):
                    _issue_gather(j)

            bg = c * bpc + b
            pltpu.sync_copy(agg_sh.at[my], agg_hbm.at[bg, my])
            _zero_my_slice()
            return 0

        lax.fori_loop(0, bpc, _batch, 0)

    return _sc


SPLIT = 12

_sc_first = _make_sc(SPLIT, 0, True)
_sc_second = _make_sc(BATCH - SPLIT, SPLIT, False)


BN = 5000


def _dense_body(x_ref, agg_ref, deg_ref, ws_ref, wn_ref, b_ref, g_ref, be_ref,
                o_ref):
    xb = x_ref[0]
    inv = 1.0 / jnp.maximum(deg_ref[0] + deg_ref[1], 1.0)
    neigh = agg_ref[0] * inv
    out = (jnp.dot(xb, ws_ref[...], preferred_element_type=jnp.float32)
           + jnp.dot(neigh, wn_ref[...], preferred_element_type=jnp.float32)
           + b_ref[...])
    mu = jnp.mean(out, axis=-1, keepdims=True)
    var = jnp.mean((out - mu) ** 2, axis=-1, keepdims=True)
    out = (out - mu) * lax.rsqrt(var + 1e-5) * g_ref[...] + be_ref[...]
    o_ref[0] = jnp.maximum(out, 0.0)


def _dense_body_aliased(x_ref, agg_ref, deg_ref, ws_ref, wn_ref, b_ref, g_ref,
                        be_ref, prev_ref, o_ref):
    del prev_ref
    _dense_body(x_ref, agg_ref, deg_ref, ws_ref, wn_ref, b_ref, g_ref, be_ref,
                o_ref)


def _make_dense(boff, nb, aliased):
    in_specs = [
        pl.BlockSpec((1, BN, D), lambda j, b: (b + boff, j, 0)),
        pl.BlockSpec((1, BN, D), lambda j, b: (b, j, 0)),
        pl.BlockSpec((NC, BN, D), lambda j, b: (0, j, 0)),
        pl.BlockSpec((D, D), lambda j, b: (0, 0)),
        pl.BlockSpec((D, D), lambda j, b: (0, 0)),
        pl.BlockSpec((1, D), lambda j, b: (0, 0)),
        pl.BlockSpec((1, D), lambda j, b: (0, 0)),
        pl.BlockSpec((1, D), lambda j, b: (0, 0)),
    ]
    body = _dense_body
    aliases = {}
    if aliased:
        in_specs.append(pl.BlockSpec(memory_space=pl.ANY))
        body = _dense_body_aliased
        aliases = {8: 0}
    return pl.pallas_call(
        body,
        grid=(N // BN, nb),
        in_specs=in_specs,
        out_specs=pl.BlockSpec((1, BN, D), lambda j, b: (b + boff, j, 0)),
        out_shape=jax.ShapeDtypeStruct((BATCH, N, D), jnp.float32),
        input_output_aliases=aliases,
        compiler_params=pltpu.CompilerParams(
            dimension_semantics=("parallel", "parallel")),
    )


_dense_a = _make_dense(0, SPLIT, False)
_dense_b = _make_dense(SPLIT, BATCH - SPLIT, True)


def kernel(x, edge_index, batch_size, W_self, W_neigh, bias, gamma, beta):
    x_flat = x.reshape(BATCH * N, D)
    src2d = edge_index[0].reshape(E // CHUNK, CHUNK)
    dst2d = edge_index[1].reshape(E // CHUNK, CHUNK)
    b2 = bias.reshape(1, D)
    g2 = gamma.reshape(1, D)
    be2 = beta.reshape(1, D)
    agg0, deg_pad = _sc_first(x_flat, src2d, dst2d)
    agg1 = _sc_second(x_flat, src2d, dst2d)
    half0 = _dense_a(x, agg0, deg_pad, W_self, W_neigh, b2, g2, be2)
    out = _dense_b(x, agg1, deg_pad, W_self, W_neigh, b2, g2, be2, half0)
    return out
